# Initial kernel scaffold; baseline (speedup 1.0000x reference)
import jax
import jax.numpy as jnp
from jax.experimental import pallas as pl


def kernel(x, Lu, Ld, har_w1, har_b1, sol_w1, sol_b1, irr_w1, irr_b1, har_w2, har_b2, sol_w2, sol_b2, irr_w2, irr_b2):
    raise NotImplementedError("write your pallas kernel here")


# speedup vs baseline: 11.8721x; 11.8721x over previous
import functools

import jax
import jax.numpy as jnp
from jax import lax
from jax.experimental import pallas as pl
from jax.experimental.pallas import tpu as pltpu
from jax.experimental.pallas import tpu_sc as plsc

N = 10000
E = 320000
D = 128
NS = 16
NPAD = 10240
ROWS_PER_TILE = NPAD // NS
EDGES_PER_TILE = E // NS
K = 128
NFULL = EDGES_PER_TILE // K
TAIL = EDGES_PER_TILE - NFULL * K
KD = 80
BM = 512

_mesh = plsc.VectorSubcoreMesh(core_axis_name="c", subcore_axis_name="s",
                               num_cores=2, num_subcores=NS)


@functools.partial(
    pl.kernel,
    out_type=(
        jax.ShapeDtypeStruct((NPAD, D), jnp.float32),
        jax.ShapeDtypeStruct((NPAD, D), jnp.float32),
    ),
    mesh=_mesh,
    scratch_types=[
        pltpu.VMEM((KD, D), jnp.float32),
        pltpu.VMEM((KD,), jnp.int32),
        pltpu.VMEM((K, D), jnp.float32),
        pltpu.VMEM_SHARED((NPAD, D), jnp.float32),
    ],
)
def _deg_kernel(dstu_hbm, dstd_hbm, degu_hbm, degd_hbm, ones_v, idx_v, zb_v, accum):
    cid = lax.axis_index("c")
    tid = lax.axis_index("s")

    @pl.loop(0, K)
    def _fill_z(i):
        for j in range(D // 16):
            zb_v[i, pl.ds(j * 16, 16)] = jnp.zeros((16,), jnp.float32)

    @pl.loop(0, KD)
    def _fill_o(i):
        for j in range(D // 16):
            ones_v[i, pl.ds(j * 16, 16)] = jnp.ones((16,), jnp.float32)

    @pl.loop(0, ROWS_PER_TILE // K)
    def _zero(i):
        off = tid * ROWS_PER_TILE + i * K
        pltpu.sync_copy(zb_v, accum.at[pl.ds(off, K), :])

    sl = pl.ds(tid * ROWS_PER_TILE, ROWS_PER_TILE)
    plsc.subcore_barrier()

    def histogram(dst_hbm):
        @pl.loop(0, EDGES_PER_TILE // KD)
        def _chunk(g):
            base = pl.multiple_of(tid * EDGES_PER_TILE + g * KD, 8)
            pltpu.sync_copy(dst_hbm.at[pl.ds(base, KD)], idx_v)
            pltpu.sync_copy(ones_v, accum.at[idx_v], add=True)

    @pl.when(cid == 0)
    def _():
        histogram(dstu_hbm)

    @pl.when(cid == 1)
    def _():
        histogram(dstd_hbm)

    plsc.subcore_barrier()

    @pl.when(cid == 0)
    def _():
        pltpu.sync_copy(accum.at[sl, :], degu_hbm.at[sl, :])

    @pl.when(cid == 1)
    def _():
        pltpu.sync_copy(accum.at[sl, :], degd_hbm.at[sl, :])


@functools.partial(
    pl.kernel,
    out_type=(
        jax.ShapeDtypeStruct((NPAD, D), jnp.float32),
        jax.ShapeDtypeStruct((NPAD, D), jnp.float32),
    ),
    mesh=_mesh,
    scratch_types=[
        pltpu.VMEM((K,), jnp.int32),
        pltpu.VMEM((K,), jnp.int32),
        pltpu.VMEM((K, D), jnp.float32),
        pltpu.VMEM((TAIL,), jnp.int32),
        pltpu.VMEM((TAIL,), jnp.int32),
        pltpu.VMEM((TAIL, D), jnp.float32),
        pltpu.VMEM((K, D), jnp.float32),
        pltpu.SemaphoreType.DMA,
        pltpu.VMEM_SHARED((NPAD, D), jnp.float32),
    ],
)
def _conv_kernel(hs_hbm, hi_hbm, srcu_hbm, dstu_hbm, srcd_hbm, dstd_hbm,
                 cs_hbm, ci_hbm,
                 sidx, didx, rows, sidxt, didxt, rowst, zb_v, gsem, accum):
    cid = lax.axis_index("c")
    tid = lax.axis_index("s")

    @pl.loop(0, K)
    def _fill_z(i):
        for j in range(D // 16):
            zb_v[i, pl.ds(j * 16, 16)] = jnp.zeros((16,), jnp.float32)

    @pl.loop(0, ROWS_PER_TILE // K)
    def _zero(i):
        off = tid * ROWS_PER_TILE + i * K
        pltpu.sync_copy(zb_v, accum.at[pl.ds(off, K), :])

    plsc.subcore_barrier()

    def conv(src_hbm, dst_hbm, h_hbm):
        ebase = tid * EDGES_PER_TILE

        @pl.loop(0, NFULL)
        def _chunk(g):
            base = pl.multiple_of(ebase + g * K, 8)
            pltpu.sync_copy(src_hbm.at[pl.ds(base, K)], sidx)
            pltpu.sync_copy(dst_hbm.at[pl.ds(base, K)], didx)
            pltpu.async_copy(h_hbm.at[sidx], rows, gsem).wait()
            pltpu.sync_copy(rows, accum.at[didx], add=True)

        tbase = pl.multiple_of(ebase + NFULL * K, 8)
        pltpu.sync_copy(src_hbm.at[pl.ds(tbase, TAIL)], sidxt)
        pltpu.sync_copy(dst_hbm.at[pl.ds(tbase, TAIL)], didxt)
        pltpu.async_copy(h_hbm.at[sidxt], rowst, gsem).wait()
        pltpu.sync_copy(rowst, accum.at[didxt], add=True)

    @pl.when(cid == 0)
    def _():
        conv(srcu_hbm, dstu_hbm, hs_hbm)

    @pl.when(cid == 1)
    def _():
        conv(srcd_hbm, dstd_hbm, hi_hbm)

    plsc.subcore_barrier()
    sl = pl.ds(tid * ROWS_PER_TILE, ROWS_PER_TILE)

    @pl.when(cid == 0)
    def _():
        pltpu.sync_copy(accum.at[sl, :], cs_hbm.at[sl, :])

    @pl.when(cid == 1)
    def _():
        pltpu.sync_copy(accum.at[sl, :], ci_hbm.at[sl, :])


def _dinv(deg_col):
    return jnp.where(deg_col > 0, lax.rsqrt(jnp.maximum(deg_col, 1e-12)), 0.0)


def _mm_body(x_ref, hw_ref, sw_ref, iw_ref, degu_ref, degd_ref,
             hh_ref, hs_ref, hi_ref):
    x = x_ref[...]
    du = _dinv(degu_ref[:, 0:1])
    dd = _dinv(degd_ref[:, 0:1])
    hh_ref[...] = lax.dot_general(x, hw_ref[...], (((1,), (1,)), ((), ())),
                                  preferred_element_type=jnp.float32)
    hs_ref[...] = du * lax.dot_general(x, sw_ref[...], (((1,), (0,)), ((), ())),
                                       preferred_element_type=jnp.float32)
    hi_ref[...] = dd * lax.dot_general(x, iw_ref[...], (((1,), (0,)), ((), ())),
                                       preferred_element_type=jnp.float32)


def _combine_mm_body(hh1_ref, cs_ref, ci_ref, degu_ref, degd_ref,
                     hb_ref, sb_ref, ib_ref, hw2_ref, sw2_ref, iw2_ref,
                     hh_ref, hs_ref, hi_ref):
    du = _dinv(degu_ref[:, 0:1])
    dd = _dinv(degd_ref[:, 0:1])
    x2 = jnp.maximum(
        hh1_ref[...] + hb_ref[...] + du * cs_ref[...] + sb_ref[...]
        + dd * ci_ref[...] + ib_ref[...], 0.0)
    hh_ref[...] = lax.dot_general(x2, hw2_ref[...], (((1,), (1,)), ((), ())),
                                  preferred_element_type=jnp.float32)
    hs_ref[...] = du * lax.dot_general(x2, sw2_ref[...], (((1,), (0,)), ((), ())),
                                       preferred_element_type=jnp.float32)
    hi_ref[...] = dd * lax.dot_general(x2, iw2_ref[...], (((1,), (0,)), ((), ())),
                                       preferred_element_type=jnp.float32)


def _combine_body(hh_ref, cs_ref, ci_ref, degu_ref, degd_ref,
                  hb_ref, sb_ref, ib_ref, out_ref):
    du = _dinv(degu_ref[:, 0:1])
    dd = _dinv(degd_ref[:, 0:1])
    out_ref[...] = jnp.maximum(
        hh_ref[...] + hb_ref[...] + du * cs_ref[...] + sb_ref[...]
        + dd * ci_ref[...] + ib_ref[...], 0.0)


_row_spec = pl.BlockSpec((BM, D), lambda i: (i, 0))
_deg_spec = pl.BlockSpec((BM, D), lambda i: (i, 0))
_w_spec = pl.BlockSpec((D, D), lambda i: (0, 0))
_b_spec = pl.BlockSpec((1, D), lambda i: (0, 0))
_GRID = (NPAD // BM,)
_sds = jax.ShapeDtypeStruct((NPAD, D), jnp.float32)

_mm_call = pl.pallas_call(
    _mm_body,
    grid=_GRID,
    in_specs=[_row_spec, _w_spec, _w_spec, _w_spec, _deg_spec, _deg_spec],
    out_specs=[_row_spec, _row_spec, _row_spec],
    out_shape=[_sds, _sds, _sds],
)

_combine_mm_call = pl.pallas_call(
    _combine_mm_body,
    grid=_GRID,
    in_specs=[_row_spec, _row_spec, _row_spec, _deg_spec, _deg_spec,
              _b_spec, _b_spec, _b_spec, _w_spec, _w_spec, _w_spec],
    out_specs=[_row_spec, _row_spec, _row_spec],
    out_shape=[_sds, _sds, _sds],
)

_combine_call = pl.pallas_call(
    _combine_body,
    grid=_GRID,
    in_specs=[_row_spec, _row_spec, _row_spec, _deg_spec, _deg_spec,
              _b_spec, _b_spec, _b_spec],
    out_specs=_row_spec,
    out_shape=_sds,
)


def kernel(x, Lu, Ld, har_w1, har_b1, sol_w1, sol_b1, irr_w1, irr_b1,
           har_w2, har_b2, sol_w2, sol_b2, irr_w2, irr_b2):
    xp = jnp.pad(x, ((0, NPAD - N), (0, 0)))
    srcu, dstu = Lu[0], Lu[1]
    srcd, dstd = Ld[0], Ld[1]
    hb1, sb1, ib1 = har_b1[None, :], sol_b1[None, :], irr_b1[None, :]
    hb2, sb2, ib2 = har_b2[None, :], sol_b2[None, :], irr_b2[None, :]

    degu, degd = _deg_kernel(dstu, dstd)
    hh1, hs1, hi1 = _mm_call(xp, har_w1, sol_w1, irr_w1, degu, degd)
    cs1, ci1 = _conv_kernel(hs1, hi1, srcu, dstu, srcd, dstd)
    hh2, hs2, hi2 = _combine_mm_call(hh1, cs1, ci1, degu, degd,
                                     hb1, sb1, ib1, har_w2, sol_w2, irr_w2)
    cs2, ci2 = _conv_kernel(hs2, hi2, srcu, dstu, srcd, dstd)
    out = _combine_call(hh2, cs2, ci2, degu, degd, hb2, sb2, ib2)
    return out[:N]


# speedup vs baseline: 20.5847x; 1.7339x over previous
import functools

import jax
import jax.numpy as jnp
from jax import lax
from jax.experimental import pallas as pl
from jax.experimental.pallas import tpu as pltpu
from jax.experimental.pallas import tpu_sc as plsc

N = 10000
E = 320000
D = 128
NS = 16
NPAD = 10240
ROWS_PER_TILE = NPAD // NS
EPT = E // NS
K = 128
NCHUNK = (EPT + K - 1) // K
EPT_P = NCHUNK * K
NBUF = 4
BM = 512

_mesh = plsc.VectorSubcoreMesh(core_axis_name="c", subcore_axis_name="s",
                               num_cores=2, num_subcores=NS)


def _zero_fill(buf):
    @pl.loop(0, K)
    def _f(i):
        for j in range(D // 16):
            buf[i, pl.ds(j * 16, 16)] = jnp.zeros((16,), jnp.float32)


def _zero_accum(accum, zb, tid):
    @pl.loop(0, ROWS_PER_TILE // K)
    def _z(i):
        off = tid * ROWS_PER_TILE + i * K
        pltpu.sync_copy(zb, accum.at[pl.ds(off, K), :])


@functools.partial(
    pl.kernel,
    out_type=(
        jax.ShapeDtypeStruct((NPAD, D), jnp.float32),
        jax.ShapeDtypeStruct((NPAD, D), jnp.float32),
    ),
    mesh=_mesh,
    scratch_types=[
        pltpu.VMEM((K, D), jnp.float32),
        pltpu.VMEM((NCHUNK, K), jnp.int32),
        pltpu.SemaphoreType.DMA,
        pltpu.VMEM_SHARED((NPAD, D), jnp.float32),
    ],
)
def _deg_kernel(dstu_hbm, dstd_hbm, degu_hbm, degd_hbm,
                ones_v, didx, sem, accum):
    cid = lax.axis_index("c")
    tid = lax.axis_index("s")

    _zero_fill(ones_v)
    _zero_accum(accum, ones_v, tid)

    @pl.loop(0, K)
    def _fill_o(i):
        for j in range(D // 16):
            ones_v[i, pl.ds(j * 16, 16)] = jnp.ones((16,), jnp.float32)

    @pl.when(cid == 0)
    def _():
        pltpu.sync_copy(dstu_hbm.at[tid], didx)

    @pl.when(cid == 1)
    def _():
        pltpu.sync_copy(dstd_hbm.at[tid], didx)

    plsc.subcore_barrier()

    WIN = 16

    @pl.loop(0, NCHUNK)
    def _chunk(g):
        pltpu.async_copy(ones_v, accum.at[didx.at[g]], sem, add=True)

        @pl.when(g >= WIN)
        def _():
            pltpu.make_async_copy(ones_v, accum.at[didx.at[0]], sem).wait()

    @pl.loop(0, WIN)
    def _drain(g):
        pltpu.make_async_copy(ones_v, accum.at[didx.at[0]], sem).wait()

    plsc.subcore_barrier()
    sl = pl.ds(tid * ROWS_PER_TILE, ROWS_PER_TILE)

    @pl.when(cid == 0)
    def _():
        pltpu.sync_copy(accum.at[sl, :], degu_hbm.at[sl, :])

    @pl.when(cid == 1)
    def _():
        pltpu.sync_copy(accum.at[sl, :], degd_hbm.at[sl, :])


RB = 2
ID = 4


@functools.partial(
    pl.kernel,
    out_type=(
        jax.ShapeDtypeStruct((NPAD, D), jnp.float32),
        jax.ShapeDtypeStruct((NPAD, D), jnp.float32),
    ),
    mesh=_mesh,
    scratch_types=[
        [pltpu.VMEM((K,), jnp.int32)] * ID,
        [pltpu.VMEM((K,), jnp.int32)] * ID,
        [pltpu.VMEM((K, D), jnp.float32)] * RB,
        [pltpu.SemaphoreType.DMA] * ID,
        [pltpu.SemaphoreType.DMA] * RB,
        [pltpu.SemaphoreType.DMA] * RB,
        pltpu.VMEM_SHARED((NPAD, D), jnp.float32),
    ],
)
def _conv_kernel(hs_hbm, hi_hbm, srcu_hbm, dstu_hbm, srcd_hbm, dstd_hbm,
                 cs_hbm, ci_hbm,
                 sidx, didx, rows, isem, gsem, ssem, accum):
    cid = lax.axis_index("c")
    tid = lax.axis_index("s")

    _zero_fill(rows[0])
    _zero_accum(accum, rows[0], tid)

    def run(src_hbm, dst_hbm, h_hbm):
        def stage_idx(c, j):
            pltpu.async_copy(src_hbm.at[tid, c], sidx[j], isem[j])
            pltpu.async_copy(dst_hbm.at[tid, c], didx[j], isem[j])

        def wait_idx(j):
            pltpu.make_async_copy(src_hbm.at[tid, 0], sidx[j], isem[j]).wait()
            pltpu.make_async_copy(src_hbm.at[tid, 0], didx[j], isem[j]).wait()

        def fire_gather(c, j, b):
            pltpu.async_copy(h_hbm.at[sidx[j]], rows[b], gsem[b])

        def wait_gather(b):
            pltpu.make_async_copy(h_hbm.at[sidx[0]], rows[b], gsem[b]).wait()

        def fire_scatter(j, b):
            pltpu.async_copy(rows[b], accum.at[didx[j]], ssem[b], add=True)

        def wait_scatter(b):
            pltpu.make_async_copy(rows[b], accum.at[didx[0]], ssem[b]).wait()

        for j in range(3):
            stage_idx(j, j)
        wait_idx(0)
        fire_gather(0, 0, 0)

        plsc.subcore_barrier()

        def body(c, k):
            j = k % ID
            b = k % RB

            @pl.when(c + 1 < NCHUNK)
            def _():
                @pl.when(c >= 1)
                def _():
                    wait_scatter((k + 1) % RB)
                wait_idx((k + 1) % ID)
                fire_gather(c + 1, (k + 1) % ID, (k + 1) % RB)

            @pl.when(c + 3 < NCHUNK)
            def _():
                stage_idx(c + 3, (k + 3) % ID)

            wait_gather(b)
            fire_scatter(j, b)

        G = NCHUNK // ID

        @pl.loop(0, G)
        def _steady(g):
            for k in range(ID):
                body(g * ID + k, k)

        for c in range(G * ID, NCHUNK):
            body(c, c % ID)
        for b in range(RB):
            wait_scatter(b)

    @pl.when(cid == 0)
    def _():
        run(srcu_hbm, dstu_hbm, hs_hbm)

    @pl.when(cid == 1)
    def _():
        run(srcd_hbm, dstd_hbm, hi_hbm)

    plsc.subcore_barrier()
    sl = pl.ds(tid * ROWS_PER_TILE, ROWS_PER_TILE)

    @pl.when(cid == 0)
    def _():
        pltpu.sync_copy(accum.at[sl, :], cs_hbm.at[sl, :])

    @pl.when(cid == 1)
    def _():
        pltpu.sync_copy(accum.at[sl, :], ci_hbm.at[sl, :])


def _dinv(deg_col):
    return jnp.where(deg_col > 0, lax.rsqrt(jnp.maximum(deg_col, 1e-12)), 0.0)


def _mm_body(x_ref, hw_ref, sw_ref, iw_ref, degu_ref, degd_ref,
             hh_ref, hs_ref, hi_ref):
    x = x_ref[...]
    du = _dinv(degu_ref[:, 0:1])
    dd = _dinv(degd_ref[:, 0:1])
    hh_ref[...] = lax.dot_general(x, hw_ref[...], (((1,), (1,)), ((), ())),
                                  preferred_element_type=jnp.float32)
    hs_ref[...] = du * lax.dot_general(x, sw_ref[...], (((1,), (0,)), ((), ())),
                                       preferred_element_type=jnp.float32)
    hi_ref[...] = dd * lax.dot_general(x, iw_ref[...], (((1,), (0,)), ((), ())),
                                       preferred_element_type=jnp.float32)


def _combine_mm_body(hh1_ref, cs_ref, ci_ref, degu_ref, degd_ref,
                     hb_ref, sb_ref, ib_ref, hw2_ref, sw2_ref, iw2_ref,
                     hh_ref, hs_ref, hi_ref):
    du = _dinv(degu_ref[:, 0:1])
    dd = _dinv(degd_ref[:, 0:1])
    x2 = jnp.maximum(
        hh1_ref[...] + hb_ref[...] + du * cs_ref[...] + sb_ref[...]
        + dd * ci_ref[...] + ib_ref[...], 0.0)
    hh_ref[...] = lax.dot_general(x2, hw2_ref[...], (((1,), (1,)), ((), ())),
                                  preferred_element_type=jnp.float32)
    hs_ref[...] = du * lax.dot_general(x2, sw2_ref[...], (((1,), (0,)), ((), ())),
                                       preferred_element_type=jnp.float32)
    hi_ref[...] = dd * lax.dot_general(x2, iw2_ref[...], (((1,), (0,)), ((), ())),
                                       preferred_element_type=jnp.float32)


def _combine_body(hh_ref, cs_ref, ci_ref, degu_ref, degd_ref,
                  hb_ref, sb_ref, ib_ref, out_ref):
    du = _dinv(degu_ref[:, 0:1])
    dd = _dinv(degd_ref[:, 0:1])
    out_ref[...] = jnp.maximum(
        hh_ref[...] + hb_ref[...] + du * cs_ref[...] + sb_ref[...]
        + dd * ci_ref[...] + ib_ref[...], 0.0)


_row_spec = pl.BlockSpec((BM, D), lambda i: (i, 0))
_deg_spec = pl.BlockSpec((BM, D), lambda i: (i, 0))
_w_spec = pl.BlockSpec((D, D), lambda i: (0, 0))
_b_spec = pl.BlockSpec((1, D), lambda i: (0, 0))
_GRID = (NPAD // BM,)
_sds = jax.ShapeDtypeStruct((NPAD, D), jnp.float32)

_mm_call = pl.pallas_call(
    _mm_body,
    grid=_GRID,
    in_specs=[_row_spec, _w_spec, _w_spec, _w_spec, _deg_spec, _deg_spec],
    out_specs=[_row_spec, _row_spec, _row_spec],
    out_shape=[_sds, _sds, _sds],
)

_combine_mm_call = pl.pallas_call(
    _combine_mm_body,
    grid=_GRID,
    in_specs=[_row_spec, _row_spec, _row_spec, _deg_spec, _deg_spec,
              _b_spec, _b_spec, _b_spec, _w_spec, _w_spec, _w_spec],
    out_specs=[_row_spec, _row_spec, _row_spec],
    out_shape=[_sds, _sds, _sds],
)

_combine_call = pl.pallas_call(
    _combine_body,
    grid=_GRID,
    in_specs=[_row_spec, _row_spec, _row_spec, _deg_spec, _deg_spec,
              _b_spec, _b_spec, _b_spec],
    out_specs=_row_spec,
    out_shape=_sds,
)


def _pad_edges(row, fill):
    r = row.reshape(NS, EPT)
    r = jnp.pad(r, ((0, 0), (0, EPT_P - EPT)), constant_values=fill)
    return r.reshape(NS, NCHUNK, K)


def kernel(x, Lu, Ld, har_w1, har_b1, sol_w1, sol_b1, irr_w1, irr_b1,
           har_w2, har_b2, sol_w2, sol_b2, irr_w2, irr_b2):
    xp = jnp.pad(x, ((0, NPAD - N), (0, 0)))
    srcu = _pad_edges(Lu[0], 0)
    dstu = _pad_edges(Lu[1], NPAD - 1)
    srcd = _pad_edges(Ld[0], 0)
    dstd = _pad_edges(Ld[1], NPAD - 1)
    hb1, sb1, ib1 = har_b1[None, :], sol_b1[None, :], irr_b1[None, :]
    hb2, sb2, ib2 = har_b2[None, :], sol_b2[None, :], irr_b2[None, :]

    degu, degd = _deg_kernel(dstu, dstd)
    hh1, hs1, hi1 = _mm_call(xp, har_w1, sol_w1, irr_w1, degu, degd)
    cs1, ci1 = _conv_kernel(hs1, hi1, srcu, dstu, srcd, dstd)
    hh2, hs2, hi2 = _combine_mm_call(hh1, cs1, ci1, degu, degd,
                                     hb1, sb1, ib1, har_w2, sol_w2, irr_w2)
    cs2, ci2 = _conv_kernel(hs2, hi2, srcu, dstu, srcd, dstd)
    out = _combine_call(hh2, cs2, ci2, degu, degd, hb2, sb2, ib2)
    return out[:N]


# speedup vs baseline: 21.4384x; 1.0415x over previous
import functools

import jax
import jax.numpy as jnp
from jax import lax
from jax.experimental import pallas as pl
from jax.experimental.pallas import tpu as pltpu
from jax.experimental.pallas import tpu_sc as plsc

N = 10000
E = 320000
D = 128
NS = 16
NPAD = 10240
ROWS_PER_TILE = NPAD // NS
EPT = E // NS
K = 128
NCHUNK = (EPT + K - 1) // K
EPT_P = NCHUNK * K
NACC = 10112
ACC_RPT = NACC // NS
BM = 512

_mesh = plsc.VectorSubcoreMesh(core_axis_name="c", subcore_axis_name="s",
                               num_cores=2, num_subcores=NS)


def _zero_fill(buf):
    @pl.loop(0, K)
    def _f(i):
        for j in range(D // 16):
            buf[i, pl.ds(j * 16, 16)] = jnp.zeros((16,), jnp.float32)


def _zero_accum(accum, zb, tid):
    base = tid * ACC_RPT
    for i in range(ACC_RPT // K):
        pltpu.sync_copy(zb, accum.at[pl.ds(base + i * K, K), :])
    rem = ACC_RPT % K
    pltpu.sync_copy(zb.at[pl.ds(0, rem), :],
                    accum.at[pl.ds(base + ACC_RPT - rem, rem), :])


@functools.partial(
    pl.kernel,
    out_type=(
        jax.ShapeDtypeStruct((NPAD, D), jnp.float32),
        jax.ShapeDtypeStruct((NPAD, D), jnp.float32),
    ),
    mesh=_mesh,
    scratch_types=[
        pltpu.VMEM((K, D), jnp.float32),
        pltpu.VMEM((NCHUNK, K), jnp.int32),
        pltpu.SemaphoreType.DMA,
        pltpu.VMEM_SHARED((NACC, D), jnp.float32),
    ],
)
def _deg_kernel(dstu_hbm, dstd_hbm, degu_hbm, degd_hbm,
                ones_v, didx, sem, accum):
    cid = lax.axis_index("c")
    tid = lax.axis_index("s")

    _zero_fill(ones_v)
    _zero_accum(accum, ones_v, tid)

    @pl.loop(0, K)
    def _fill_o(i):
        for j in range(D // 16):
            ones_v[i, pl.ds(j * 16, 16)] = jnp.ones((16,), jnp.float32)

    @pl.when(cid == 0)
    def _():
        pltpu.sync_copy(dstu_hbm.at[tid], didx)

    @pl.when(cid == 1)
    def _():
        pltpu.sync_copy(dstd_hbm.at[tid], didx)

    plsc.subcore_barrier()

    WIN = 16

    @pl.loop(0, NCHUNK)
    def _chunk(g):
        pltpu.async_copy(ones_v, accum.at[didx.at[g]], sem, add=True)

        @pl.when(g >= WIN)
        def _():
            pltpu.make_async_copy(ones_v, accum.at[didx.at[0]], sem).wait()

    @pl.loop(0, WIN)
    def _drain(g):
        pltpu.make_async_copy(ones_v, accum.at[didx.at[0]], sem).wait()

    plsc.subcore_barrier()
    sl = pl.ds(tid * ACC_RPT, ACC_RPT)

    @pl.when(cid == 0)
    def _():
        pltpu.sync_copy(accum.at[sl, :], degu_hbm.at[sl, :])

    @pl.when(cid == 1)
    def _():
        pltpu.sync_copy(accum.at[sl, :], degd_hbm.at[sl, :])


RB = 3
ID = 4


@functools.partial(
    pl.kernel,
    out_type=(
        jax.ShapeDtypeStruct((NPAD, D), jnp.float32),
        jax.ShapeDtypeStruct((NPAD, D), jnp.float32),
    ),
    mesh=_mesh,
    scratch_types=[
        [pltpu.VMEM((K,), jnp.int32)] * ID,
        [pltpu.VMEM((K,), jnp.int32)] * ID,
        [pltpu.VMEM((K, D), jnp.float32)] * RB,
        [pltpu.SemaphoreType.DMA] * ID,
        [pltpu.SemaphoreType.DMA] * RB,
        [pltpu.SemaphoreType.DMA] * RB,
        pltpu.VMEM_SHARED((NACC, D), jnp.float32),
    ],
)
def _conv_kernel(hs_hbm, hi_hbm, srcu_hbm, dstu_hbm, srcd_hbm, dstd_hbm,
                 cs_hbm, ci_hbm,
                 sidx, didx, rows, isem, gsem, ssem, accum):
    cid = lax.axis_index("c")
    tid = lax.axis_index("s")

    _zero_fill(rows[0])
    _zero_accum(accum, rows[0], tid)

    def run(src_hbm, dst_hbm, h_hbm):
        def stage_idx(c, j):
            pltpu.async_copy(src_hbm.at[tid, c], sidx[j], isem[j])
            pltpu.async_copy(dst_hbm.at[tid, c], didx[j], isem[j])

        def wait_idx(j):
            pltpu.make_async_copy(src_hbm.at[tid, 0], sidx[j], isem[j]).wait()
            pltpu.make_async_copy(src_hbm.at[tid, 0], didx[j], isem[j]).wait()

        def fire_gather(j, b):
            pltpu.async_copy(h_hbm.at[sidx[j]], rows[b], gsem[b])

        def wait_gather(b):
            pltpu.make_async_copy(h_hbm.at[sidx[0]], rows[b], gsem[b]).wait()

        def fire_scatter(j, b):
            pltpu.async_copy(rows[b], accum.at[didx[j]], ssem[b], add=True)

        def wait_scatter(b):
            pltpu.make_async_copy(rows[b], accum.at[didx[0]], ssem[b]).wait()

        for j in range(2):
            stage_idx(j, j)
        wait_idx(0)
        fire_gather(0, 0)

        plsc.subcore_barrier()

        def body(c, k):
            j = k % ID
            b = k % RB

            @pl.when(c + 1 < NCHUNK)
            def _():
                @pl.when(c >= RB - 1)
                def _():
                    wait_scatter((k + 1) % RB)
                wait_idx((k + 1) % ID)
                fire_gather((k + 1) % ID, (k + 1) % RB)

            @pl.when(c + 2 < NCHUNK)
            def _():
                stage_idx(c + 2, (k + 2) % ID)

            wait_gather(b)
            fire_scatter(j, b)

        SUP = 12
        G = NCHUNK // SUP

        @pl.loop(0, G)
        def _steady(g):
            for k in range(SUP):
                body(g * SUP + k, k)

        for c in range(G * SUP, NCHUNK):
            body(c, c % SUP)
        for b in range(RB):
            wait_scatter(b)

    @pl.when(cid == 0)
    def _():
        run(srcu_hbm, dstu_hbm, hs_hbm)

    @pl.when(cid == 1)
    def _():
        run(srcd_hbm, dstd_hbm, hi_hbm)

    plsc.subcore_barrier()
    sl = pl.ds(tid * ACC_RPT, ACC_RPT)

    @pl.when(cid == 0)
    def _():
        pltpu.sync_copy(accum.at[sl, :], cs_hbm.at[sl, :])

    @pl.when(cid == 1)
    def _():
        pltpu.sync_copy(accum.at[sl, :], ci_hbm.at[sl, :])


def _dinv(deg_col):
    return jnp.where(deg_col > 0, lax.rsqrt(jnp.maximum(deg_col, 1e-12)), 0.0)


def _mm_body(x_ref, hw_ref, sw_ref, iw_ref, degu_ref, degd_ref,
             hh_ref, hs_ref, hi_ref):
    x = x_ref[...]
    du = _dinv(degu_ref[:, 0:1])
    dd = _dinv(degd_ref[:, 0:1])
    hh_ref[...] = lax.dot_general(x, hw_ref[...], (((1,), (1,)), ((), ())),
                                  preferred_element_type=jnp.float32)
    hs_ref[...] = du * lax.dot_general(x, sw_ref[...], (((1,), (0,)), ((), ())),
                                       preferred_element_type=jnp.float32)
    hi_ref[...] = dd * lax.dot_general(x, iw_ref[...], (((1,), (0,)), ((), ())),
                                       preferred_element_type=jnp.float32)


def _combine_mm_body(hh1_ref, cs_ref, ci_ref, degu_ref, degd_ref,
                     hb_ref, sb_ref, ib_ref, hw2_ref, sw2_ref, iw2_ref,
                     hh_ref, hs_ref, hi_ref):
    du = _dinv(degu_ref[:, 0:1])
    dd = _dinv(degd_ref[:, 0:1])
    x2 = jnp.maximum(
        hh1_ref[...] + hb_ref[...] + du * cs_ref[...] + sb_ref[...]
        + dd * ci_ref[...] + ib_ref[...], 0.0)
    hh_ref[...] = lax.dot_general(x2, hw2_ref[...], (((1,), (1,)), ((), ())),
                                  preferred_element_type=jnp.float32)
    hs_ref[...] = du * lax.dot_general(x2, sw2_ref[...], (((1,), (0,)), ((), ())),
                                       preferred_element_type=jnp.float32)
    hi_ref[...] = dd * lax.dot_general(x2, iw2_ref[...], (((1,), (0,)), ((), ())),
                                       preferred_element_type=jnp.float32)


def _combine_body(hh_ref, cs_ref, ci_ref, degu_ref, degd_ref,
                  hb_ref, sb_ref, ib_ref, out_ref):
    du = _dinv(degu_ref[:, 0:1])
    dd = _dinv(degd_ref[:, 0:1])
    out_ref[...] = jnp.maximum(
        hh_ref[...] + hb_ref[...] + du * cs_ref[...] + sb_ref[...]
        + dd * ci_ref[...] + ib_ref[...], 0.0)


_row_spec = pl.BlockSpec((BM, D), lambda i: (i, 0))
_deg_spec = pl.BlockSpec((BM, D), lambda i: (i, 0))
_w_spec = pl.BlockSpec((D, D), lambda i: (0, 0))
_b_spec = pl.BlockSpec((1, D), lambda i: (0, 0))
_GRID = (NPAD // BM,)
_sds = jax.ShapeDtypeStruct((NPAD, D), jnp.float32)

_mm_call = pl.pallas_call(
    _mm_body,
    grid=_GRID,
    in_specs=[_row_spec, _w_spec, _w_spec, _w_spec, _deg_spec, _deg_spec],
    out_specs=[_row_spec, _row_spec, _row_spec],
    out_shape=[_sds, _sds, _sds],
)

_combine_mm_call = pl.pallas_call(
    _combine_mm_body,
    grid=_GRID,
    in_specs=[_row_spec, _row_spec, _row_spec, _deg_spec, _deg_spec,
              _b_spec, _b_spec, _b_spec, _w_spec, _w_spec, _w_spec],
    out_specs=[_row_spec, _row_spec, _row_spec],
    out_shape=[_sds, _sds, _sds],
)

_combine_call = pl.pallas_call(
    _combine_body,
    grid=_GRID,
    in_specs=[_row_spec, _row_spec, _row_spec, _deg_spec, _deg_spec,
              _b_spec, _b_spec, _b_spec],
    out_specs=_row_spec,
    out_shape=_sds,
)


def _pad_edges(row, fill):
    r = row.reshape(NS, EPT)
    r = jnp.pad(r, ((0, 0), (0, EPT_P - EPT)), constant_values=fill)
    return r.reshape(NS, NCHUNK, K)


def kernel(x, Lu, Ld, har_w1, har_b1, sol_w1, sol_b1, irr_w1, irr_b1,
           har_w2, har_b2, sol_w2, sol_b2, irr_w2, irr_b2):
    xp = jnp.pad(x, ((0, NPAD - N), (0, 0)))
    srcu = _pad_edges(Lu[0], 0)
    dstu = _pad_edges(Lu[1], N)
    srcd = _pad_edges(Ld[0], 0)
    dstd = _pad_edges(Ld[1], N)
    hb1, sb1, ib1 = har_b1[None, :], sol_b1[None, :], irr_b1[None, :]
    hb2, sb2, ib2 = har_b2[None, :], sol_b2[None, :], irr_b2[None, :]

    degu, degd = _deg_kernel(dstu, dstd)
    hh1, hs1, hi1 = _mm_call(xp, har_w1, sol_w1, irr_w1, degu, degd)
    cs1, ci1 = _conv_kernel(hs1, hi1, srcu, dstu, srcd, dstd)
    hh2, hs2, hi2 = _combine_mm_call(hh1, cs1, ci1, degu, degd,
                                     hb1, sb1, ib1, har_w2, sol_w2, irr_w2)
    cs2, ci2 = _conv_kernel(hs2, hi2, srcu, dstu, srcd, dstd)
    out = _combine_call(hh2, cs2, ci2, degu, degd, hb2, sb2, ib2)
    return out[:N]


# speedup vs baseline: 21.5183x; 1.0037x over previous
import functools

import jax
import jax.numpy as jnp
from jax import lax
from jax.experimental import pallas as pl
from jax.experimental.pallas import tpu as pltpu
from jax.experimental.pallas import tpu_sc as plsc

N = 10000
E = 320000
D = 128
NS = 16
NPAD = 10240
ROWS_PER_TILE = NPAD // NS
EPT = E // NS
K = 128
NCHUNK = (EPT + K - 1) // K
EPT_P = NCHUNK * K
NACC = 10112
ACC_RPT = NACC // NS
BM = 512

_mesh = plsc.VectorSubcoreMesh(core_axis_name="c", subcore_axis_name="s",
                               num_cores=2, num_subcores=NS)


def _zero_fill(buf):
    @pl.loop(0, K)
    def _f(i):
        for j in range(D // 16):
            buf[i, pl.ds(j * 16, 16)] = jnp.zeros((16,), jnp.float32)


def _zero_accum(accum, zb, tid):
    base = tid * ACC_RPT
    for i in range(ACC_RPT // K):
        pltpu.sync_copy(zb, accum.at[pl.ds(base + i * K, K), :])
    rem = ACC_RPT % K
    pltpu.sync_copy(zb.at[pl.ds(0, rem), :],
                    accum.at[pl.ds(base + ACC_RPT - rem, rem), :])


@functools.partial(
    pl.kernel,
    out_type=(
        jax.ShapeDtypeStruct((NPAD, D), jnp.float32),
        jax.ShapeDtypeStruct((NPAD, D), jnp.float32),
    ),
    mesh=_mesh,
    scratch_types=[
        pltpu.VMEM((K, D), jnp.float32),
        pltpu.VMEM((NCHUNK, K), jnp.int32),
        pltpu.SemaphoreType.DMA,
        pltpu.VMEM_SHARED((NACC, D), jnp.float32),
    ],
)
def _deg_kernel(dstu_hbm, dstd_hbm, degu_hbm, degd_hbm,
                ones_v, didx, sem, accum):
    cid = lax.axis_index("c")
    tid = lax.axis_index("s")

    _zero_fill(ones_v)
    _zero_accum(accum, ones_v, tid)

    @pl.loop(0, K)
    def _fill_o(i):
        for j in range(D // 16):
            ones_v[i, pl.ds(j * 16, 16)] = jnp.ones((16,), jnp.float32)

    @pl.when(cid == 0)
    def _():
        pltpu.sync_copy(dstu_hbm.at[tid], didx)

    @pl.when(cid == 1)
    def _():
        pltpu.sync_copy(dstd_hbm.at[tid], didx)

    plsc.subcore_barrier()

    WIN = 16

    @pl.loop(0, NCHUNK)
    def _chunk(g):
        pltpu.async_copy(ones_v, accum.at[didx.at[g]], sem, add=True)

        @pl.when(g >= WIN)
        def _():
            pltpu.make_async_copy(ones_v, accum.at[didx.at[0]], sem).wait()

    @pl.loop(0, WIN)
    def _drain(g):
        pltpu.make_async_copy(ones_v, accum.at[didx.at[0]], sem).wait()

    plsc.subcore_barrier()
    sl = pl.ds(tid * ACC_RPT, ACC_RPT)

    @pl.when(cid == 0)
    def _():
        pltpu.sync_copy(accum.at[sl, :], degu_hbm.at[sl, :])

    @pl.when(cid == 1)
    def _():
        pltpu.sync_copy(accum.at[sl, :], degd_hbm.at[sl, :])


RB = 3
ID = 4


@functools.partial(
    pl.kernel,
    out_type=(
        jax.ShapeDtypeStruct((NPAD, D), jnp.float32),
        jax.ShapeDtypeStruct((NPAD, D), jnp.float32),
    ),
    mesh=_mesh,
    scratch_types=[
        [pltpu.VMEM((2, K), jnp.int32)] * ID,
        [pltpu.VMEM((K, D), jnp.float32)] * RB,
        [pltpu.SemaphoreType.DMA] * ID,
        [pltpu.SemaphoreType.DMA] * RB,
        [pltpu.SemaphoreType.DMA] * RB,
        pltpu.VMEM_SHARED((NACC, D), jnp.float32),
    ],
)
def _conv_kernel(hs_hbm, hi_hbm, eu_hbm, ed_hbm, cs_hbm, ci_hbm,
                 eidx, rows, isem, gsem, ssem, accum):
    cid = lax.axis_index("c")
    tid = lax.axis_index("s")

    _zero_fill(rows[0])
    _zero_accum(accum, rows[0], tid)

    def run(e_hbm, h_hbm):
        def stage_idx(c, j):
            pltpu.async_copy(e_hbm.at[tid, c], eidx[j], isem[j])

        def wait_idx(j):
            pltpu.make_async_copy(e_hbm.at[tid, 0], eidx[j], isem[j]).wait()

        def fire_gather(j, b):
            pltpu.async_copy(h_hbm.at[eidx[j].at[0]], rows[b], gsem[b])

        def wait_gather(b):
            pltpu.make_async_copy(h_hbm.at[eidx[0].at[0]], rows[b],
                                  gsem[b]).wait()

        def fire_scatter(j, b):
            pltpu.async_copy(rows[b], accum.at[eidx[j].at[1]], ssem[b],
                             add=True)

        def wait_scatter(b):
            pltpu.make_async_copy(rows[b], accum.at[eidx[0].at[1]],
                                  ssem[b]).wait()

        for j in range(2):
            stage_idx(j, j)
        wait_idx(0)
        fire_gather(0, 0)

        plsc.subcore_barrier()

        def body(c, k):
            j = k % ID
            b = k % RB

            @pl.when(c + 1 < NCHUNK)
            def _():
                @pl.when(c >= RB - 1)
                def _():
                    wait_scatter((k + 1) % RB)
                wait_idx((k + 1) % ID)
                fire_gather((k + 1) % ID, (k + 1) % RB)

            @pl.when(c + 2 < NCHUNK)
            def _():
                stage_idx(c + 2, (k + 2) % ID)

            wait_gather(b)
            fire_scatter(j, b)

        SUP = 12
        G = NCHUNK // SUP

        @pl.loop(0, G)
        def _steady(g):
            for k in range(SUP):
                body(g * SUP + k, k)

        for c in range(G * SUP, NCHUNK):
            body(c, c % SUP)
        for b in range(RB):
            wait_scatter(b)

    @pl.when(cid == 0)
    def _():
        run(eu_hbm, hs_hbm)

    @pl.when(cid == 1)
    def _():
        run(ed_hbm, hi_hbm)

    plsc.subcore_barrier()
    sl = pl.ds(tid * ACC_RPT, ACC_RPT)

    @pl.when(cid == 0)
    def _():
        pltpu.sync_copy(accum.at[sl, :], cs_hbm.at[sl, :])

    @pl.when(cid == 1)
    def _():
        pltpu.sync_copy(accum.at[sl, :], ci_hbm.at[sl, :])


def _dinv(deg_col):
    return jnp.where(deg_col > 0, lax.rsqrt(jnp.maximum(deg_col, 1e-12)), 0.0)


def _mm_body(x_ref, hw_ref, sw_ref, iw_ref, degu_ref, degd_ref,
             hh_ref, hs_ref, hi_ref):
    x = x_ref[...]
    du = _dinv(degu_ref[:, 0:1])
    dd = _dinv(degd_ref[:, 0:1])
    hh_ref[...] = lax.dot_general(x, hw_ref[...], (((1,), (1,)), ((), ())),
                                  preferred_element_type=jnp.float32)
    hs_ref[...] = du * lax.dot_general(x, sw_ref[...], (((1,), (0,)), ((), ())),
                                       preferred_element_type=jnp.float32)
    hi_ref[...] = dd * lax.dot_general(x, iw_ref[...], (((1,), (0,)), ((), ())),
                                       preferred_element_type=jnp.float32)


def _combine_mm_body(hh1_ref, cs_ref, ci_ref, degu_ref, degd_ref,
                     hb_ref, sb_ref, ib_ref, hw2_ref, sw2_ref, iw2_ref,
                     hh_ref, hs_ref, hi_ref):
    du = _dinv(degu_ref[:, 0:1])
    dd = _dinv(degd_ref[:, 0:1])
    x2 = jnp.maximum(
        hh1_ref[...] + hb_ref[...] + du * cs_ref[...] + sb_ref[...]
        + dd * ci_ref[...] + ib_ref[...], 0.0)
    hh_ref[...] = lax.dot_general(x2, hw2_ref[...], (((1,), (1,)), ((), ())),
                                  preferred_element_type=jnp.float32)
    hs_ref[...] = du * lax.dot_general(x2, sw2_ref[...], (((1,), (0,)), ((), ())),
                                       preferred_element_type=jnp.float32)
    hi_ref[...] = dd * lax.dot_general(x2, iw2_ref[...], (((1,), (0,)), ((), ())),
                                       preferred_element_type=jnp.float32)


def _combine_body(hh_ref, cs_ref, ci_ref, degu_ref, degd_ref,
                  hb_ref, sb_ref, ib_ref, out_ref):
    du = _dinv(degu_ref[:, 0:1])
    dd = _dinv(degd_ref[:, 0:1])
    out_ref[...] = jnp.maximum(
        hh_ref[...] + hb_ref[...] + du * cs_ref[...] + sb_ref[...]
        + dd * ci_ref[...] + ib_ref[...], 0.0)


_row_spec = pl.BlockSpec((BM, D), lambda i: (i, 0))
_deg_spec = pl.BlockSpec((BM, D), lambda i: (i, 0))
_w_spec = pl.BlockSpec((D, D), lambda i: (0, 0))
_b_spec = pl.BlockSpec((1, D), lambda i: (0, 0))
_GRID = (NPAD // BM,)
_sds = jax.ShapeDtypeStruct((NPAD, D), jnp.float32)

_mm_call = pl.pallas_call(
    _mm_body,
    grid=_GRID,
    in_specs=[_row_spec, _w_spec, _w_spec, _w_spec, _deg_spec, _deg_spec],
    out_specs=[_row_spec, _row_spec, _row_spec],
    out_shape=[_sds, _sds, _sds],
)

_combine_mm_call = pl.pallas_call(
    _combine_mm_body,
    grid=_GRID,
    in_specs=[_row_spec, _row_spec, _row_spec, _deg_spec, _deg_spec,
              _b_spec, _b_spec, _b_spec, _w_spec, _w_spec, _w_spec],
    out_specs=[_row_spec, _row_spec, _row_spec],
    out_shape=[_sds, _sds, _sds],
)

_combine_call = pl.pallas_call(
    _combine_body,
    grid=_GRID,
    in_specs=[_row_spec, _row_spec, _row_spec, _deg_spec, _deg_spec,
              _b_spec, _b_spec, _b_spec],
    out_specs=_row_spec,
    out_shape=_sds,
)


def _pad_edges(row, fill):
    r = row.reshape(NS, EPT)
    r = jnp.pad(r, ((0, 0), (0, EPT_P - EPT)), constant_values=fill)
    return r.reshape(NS, NCHUNK, K)


def kernel(x, Lu, Ld, har_w1, har_b1, sol_w1, sol_b1, irr_w1, irr_b1,
           har_w2, har_b2, sol_w2, sol_b2, irr_w2, irr_b2):
    xp = jnp.pad(x, ((0, NPAD - N), (0, 0)))
    dstu = _pad_edges(Lu[1], N)
    dstd = _pad_edges(Ld[1], N)
    eu = jnp.stack([_pad_edges(Lu[0], 0), dstu], axis=2)
    ed = jnp.stack([_pad_edges(Ld[0], 0), dstd], axis=2)
    hb1, sb1, ib1 = har_b1[None, :], sol_b1[None, :], irr_b1[None, :]
    hb2, sb2, ib2 = har_b2[None, :], sol_b2[None, :], irr_b2[None, :]

    degu, degd = _deg_kernel(dstu, dstd)
    hh1, hs1, hi1 = _mm_call(xp, har_w1, sol_w1, irr_w1, degu, degd)
    cs1, ci1 = _conv_kernel(hs1, hi1, eu, ed)
    hh2, hs2, hi2 = _combine_mm_call(hh1, cs1, ci1, degu, degd,
                                     hb1, sb1, ib1, har_w2, sol_w2, irr_w2)
    cs2, ci2 = _conv_kernel(hs2, hi2, eu, ed)
    out = _combine_call(hh2, cs2, ci2, degu, degd, hb2, sb2, ib2)
    return out[:N]
